# baseline (speedup 1.0000x reference)
import functools

import jax
import jax.numpy as jnp
import numpy as np
from jax.experimental import pallas as pl
from jax.experimental.pallas import tpu as pltpu

L = 2048
GEO_DIM = 1536
SEM_DIM = 512
RSA_DIM = 64
D = 256
H = 8
DH = D // H
BQ = 256
NBLK = L // BQ


def _ln(x, g, b, eps=1e-5):
    mu = jnp.mean(x, axis=-1, keepdims=True)
    xc = x - mu
    var = jnp.mean(xc * xc, axis=-1, keepdims=True)
    return xc / jnp.sqrt(var + eps) * g + b


def _headln(x, S, B, eps=1e-5):
    mu = (x @ S) @ B
    xc = x - mu
    var = ((xc * xc) @ S) @ B
    return xc / jnp.sqrt(var + eps)


def _lrelu(x):
    return jnp.where(x >= 0, x, 0.01 * x)


def _proj_kernel(gf, sf, rf,
                 Wg, bg, gg, gb,
                 Ws, bs, sg, sb,
                 Wr, br, rg, rb,
                 Wqg, bqg, Wqs, bqs,
                 Wk, bk, Wv, bv,
                 Wt, bt, tg, tb,
                 S, B,
                 geo_p_o, sem_p_o, qg_o, qs_o, kg_o, ks_o, vg_o, vs_o, rsa_o):
    Sm, Bm = S[...], B[...]
    geo_p = _ln(gf[...] @ Wg[...] + bg[...], gg[...], gb[...])
    sem_p = _ln(sf[...] @ Ws[...] + bs[...], sg[...], sb[...])
    rsa_p = _ln(rf[...] @ Wr[...] + br[...], rg[...], rb[...])
    geo_p_o[...] = geo_p
    sem_p_o[...] = sem_p
    qg_o[...] = geo_p @ Wqg[...] + bqg[...]
    qs_o[...] = sem_p @ Wqs[...] + bqs[...]
    kg_o[...] = _headln(geo_p @ Wk[...] + bk[...], Sm, Bm)
    ks_o[...] = _headln(sem_p @ Wk[...] + bk[...], Sm, Bm)
    vg_o[...] = _headln(geo_p @ Wv[...] + bv[...], Sm, Bm)
    vs_o[...] = _headln(sem_p @ Wv[...] + bv[...], Sm, Bm)
    rsa_o[...] = _lrelu(_ln(rsa_p @ Wt[...] + bt[...], tg[...], tb[...]))


def _knn_kernel(pos_b, posT, m8_o, m16_o):
    pb = pos_b[...]
    pT = posT[...]
    d2 = jnp.zeros((BQ, L), jnp.float32)
    for c in range(3):
        diff = pb[:, c:c + 1] - pT[c:c + 1, :]
        d2 = d2 + diff * diff
    iota = jax.lax.broadcasted_iota(jnp.int32, (BQ, L), 1)
    sel = jnp.zeros((BQ, L), jnp.float32)
    cur = d2
    for t in range(16):
        v = jnp.min(cur, axis=1, keepdims=True)
        cand = jnp.where(cur == v, iota, L)
        j = jnp.min(cand, axis=1, keepdims=True)
        pick = iota == j
        sel = sel + pick.astype(jnp.float32)
        cur = jnp.where(pick, jnp.inf, cur)
        if t == 7:
            m8_o[...] = sel.astype(jnp.bfloat16)
    m16_o[...] = sel.astype(jnp.bfloat16)


def _attn_kernel(q, k_self, k_cross, v_self, v_cross, m8, m16,
                 o8_o, o16_o):
    scale = jnp.float32(1.0 / np.sqrt(DH))
    m8f = m8[...]
    m16f = m16[...]
    dn = (((1,), (1,)), ((), ()))
    bf = jnp.bfloat16
    qh = (q[0, 0] * scale).astype(bf)
    ks_m = k_self[0, 0].astype(bf)
    kc_m = k_cross[0, 0].astype(bf)
    s_self = jax.lax.dot_general(qh, ks_m, dn,
                                 preferred_element_type=jnp.float32)
    s_cross = jax.lax.dot_general(qh, kc_m, dn,
                                  preferred_element_type=jnp.float32)
    m = jnp.maximum(jnp.max(s_self, axis=1, keepdims=True),
                    jnp.max(s_cross, axis=1, keepdims=True))
    e_self = jnp.exp(s_self - m)
    e_cross = jnp.exp(s_cross - m)
    zc = jnp.sum(e_cross, axis=1, keepdims=True)
    uc = jax.lax.dot_general(e_cross.astype(bf), v_cross[0, 0].astype(bf),
                             (((1,), (0,)), ((), ())),
                             preferred_element_type=jnp.float32)
    e8 = e_self * m8f.astype(jnp.float32)
    e16 = e_self * m16f.astype(jnp.float32)
    z8 = zc + jnp.sum(e8, axis=1, keepdims=True)
    z16 = zc + jnp.sum(e16, axis=1, keepdims=True)
    vs_m = v_self[0, 0].astype(bf)
    mm = (((1,), (0,)), ((), ()))
    u8 = uc + jax.lax.dot_general(e8.astype(bf), vs_m, mm,
                                  preferred_element_type=jnp.float32)
    u16 = uc + jax.lax.dot_general(e16.astype(bf), vs_m, mm,
                                   preferred_element_type=jnp.float32)
    o8_o[0, 0] = u8 / z8
    o16_o[0, 0] = u16 / z16


def _mix_kernel(g8, g16, s8, s16, geo_p, sem_p, rsa_out,
                Wo, bo, ln1g, ln1b, ln2g, ln2b,
                Wf1, bf1, f1g, f1b, Wf2, bf2, f2g, f2b,
                mix, out_o):
    mv = mix[...]
    swg = mv[0:1, 0:2]
    sws = mv[0:1, 2:4]
    wg = jnp.exp(swg - jnp.max(swg))
    wg = wg / jnp.sum(wg)
    ws = jnp.exp(sws - jnp.max(sws))
    ws = ws / jnp.sum(ws)
    wg0, wg1 = wg[0:1, 0:1], wg[0:1, 1:2]
    ws0, ws1 = ws[0:1, 0:1], ws[0:1, 1:2]
    alpha_g = mv[0:1, 4:5]
    beta_g = mv[0:1, 5:6]
    alpha_s = mv[0:1, 6:7]
    beta_s = mv[0:1, 7:8]

    Wo_m = Wo[...]
    bo_m = bo[...]
    geo_attn = (wg0 * g8[...] + wg1 * g16[...]) @ Wo_m + bo_m
    sem_attn = (ws0 * s8[...] + ws1 * s16[...]) @ Wo_m + bo_m
    geo_out = _ln(alpha_g * geo_p[...] + beta_g * geo_attn, ln1g[...], ln1b[...])
    sem_out = _ln(alpha_s * sem_p[...] + beta_s * sem_attn, ln2g[...], ln2b[...])
    W1 = Wf1[...]
    h1 = (geo_out @ W1[0:D, :] + sem_out @ W1[D:2 * D, :]
          + rsa_out[...] @ W1[2 * D:3 * D, :] + bf1[...])
    x = _lrelu(_ln(h1, f1g[...], f1b[...]))
    x = _lrelu(_ln(x @ Wf2[...] + bf2[...], f2g[...], f2b[...]))
    out_o[...] = x


def _row(v):
    return v.reshape(1, -1)


def _full_spec(shape):
    n = len(shape)
    return pl.BlockSpec(shape, lambda i, _n=n: (0,) * _n)


def _blk_spec(cols):
    return pl.BlockSpec((BQ, cols), lambda i: (i, 0))


@jax.jit
def kernel(geo_feat, sem_feat, rsa_feat, pos, params):
    p = params
    f32 = jnp.float32

    posT = jnp.zeros((8, L), f32).at[0:3, :].set(pos.T)
    pos_pad = jnp.zeros((L, 8), f32).at[:, 0:3].set(pos)

    S = np.zeros((D, 128), np.float32)
    B = np.zeros((128, D), np.float32)
    for h in range(H):
        S[h * DH:(h + 1) * DH, h] = 1.0 / DH
        B[h, h * DH:(h + 1) * DH] = 1.0
    S = jnp.asarray(S)
    B = jnp.asarray(B)

    mix = jnp.zeros((1, 128), f32)
    mix = mix.at[0, 0:2].set(p['sw_g'])
    mix = mix.at[0, 2:4].set(p['sw_s'])
    mix = mix.at[0, 4].set(p['alpha_g'])
    mix = mix.at[0, 5].set(p['beta_g'])
    mix = mix.at[0, 6].set(p['alpha_s'])
    mix = mix.at[0, 7].set(p['beta_s'])

    LD = jax.ShapeDtypeStruct((L, D), f32)

    proj_in = [geo_feat, sem_feat, rsa_feat,
               p['Wg'], _row(p['bg']), _row(p['g_g']), _row(p['g_b']),
               p['Ws'], _row(p['bs']), _row(p['s_g']), _row(p['s_b']),
               p['Wr'], _row(p['br']), _row(p['r_g']), _row(p['r_b']),
               p['Wqg'], _row(p['bqg']), p['Wqs'], _row(p['bqs']),
               p['Wk'], _row(p['bk']), p['Wv'], _row(p['bv']),
               p['Wt'], _row(p['bt']), _row(p['t_g']), _row(p['t_b']),
               S, B]
    proj_specs = ([_blk_spec(GEO_DIM), _blk_spec(SEM_DIM), _blk_spec(RSA_DIM)]
                  + [_full_spec(a.shape) for a in proj_in[3:]])
    geo_p, sem_p, qg, qs, kg, ks, vg, vs, rsa_out = pl.pallas_call(
        _proj_kernel,
        grid=(NBLK,),
        in_specs=proj_specs,
        out_specs=[_blk_spec(D)] * 9,
        out_shape=[LD] * 9,
    )(*proj_in)

    m8, m16 = pl.pallas_call(
        _knn_kernel,
        grid=(NBLK,),
        in_specs=[_blk_spec(8), _full_spec((8, L))],
        out_specs=[_blk_spec(L)] * 2,
        out_shape=[jax.ShapeDtypeStruct((L, L), jnp.bfloat16)] * 2,
    )(pos_pad, posT)

    def _split(a, b):
        return jnp.stack([a.reshape(L, H, DH).transpose(1, 0, 2),
                          b.reshape(L, H, DH).transpose(1, 0, 2)])

    Qh = _split(qg, qs)
    Kh_self = _split(kg, ks)
    Kh_cross = _split(ks, kg)
    Vh_self = _split(vg, vs)
    Vh_cross = _split(vs, vg)

    hb = pl.BlockSpec((1, 1, BQ, DH), lambda qi, sh: (sh // H, sh % H, qi, 0))
    hf_self = pl.BlockSpec((1, 1, L, DH), lambda qi, sh: (sh // H, sh % H, 0, 0))
    mspec = pl.BlockSpec((BQ, L), lambda qi, sh: (qi, 0))
    o8h, o16h = pl.pallas_call(
        _attn_kernel,
        grid=(NBLK, 2 * H),
        in_specs=[hb, hf_self, hf_self, hf_self, hf_self, mspec, mspec],
        out_specs=[hb, hb],
        out_shape=[jax.ShapeDtypeStruct((2, H, L, DH), f32)] * 2,
    )(Qh, Kh_self, Kh_cross, Vh_self, Vh_cross, m8, m16)

    g8 = o8h[0].transpose(1, 0, 2).reshape(L, D)
    s8 = o8h[1].transpose(1, 0, 2).reshape(L, D)
    g16 = o16h[0].transpose(1, 0, 2).reshape(L, D)
    s16 = o16h[1].transpose(1, 0, 2).reshape(L, D)

    mix_in = [g8, g16, s8, s16, geo_p, sem_p, rsa_out,
              p['Wo'], _row(p['bo']),
              _row(p['ln1_g']), _row(p['ln1_b']),
              _row(p['ln2_g']), _row(p['ln2_b']),
              p['Wf1'], _row(p['bf1']), _row(p['f1_g']), _row(p['f1_b']),
              p['Wf2'], _row(p['bf2']), _row(p['f2_g']), _row(p['f2_b']),
              mix]
    mix_specs = ([_blk_spec(D)] * 7
                 + [_full_spec(a.shape) for a in mix_in[7:]])
    out = pl.pallas_call(
        _mix_kernel,
        grid=(NBLK,),
        in_specs=mix_specs,
        out_specs=_blk_spec(D),
        out_shape=LD,
    )(*mix_in)
    return out


# speedup vs baseline: 1.3455x; 1.3455x over previous
import functools

import jax
import jax.numpy as jnp
import numpy as np
from jax.experimental import pallas as pl
from jax.experimental.pallas import tpu as pltpu

L = 2048
GEO_DIM = 1536
SEM_DIM = 512
RSA_DIM = 64
D = 256
H = 8
DH = D // H
BQ = 256
NBLK = L // BQ


def _ln(x, g, b, eps=1e-5):
    mu = jnp.mean(x, axis=-1, keepdims=True)
    xc = x - mu
    var = jnp.mean(xc * xc, axis=-1, keepdims=True)
    return xc / jnp.sqrt(var + eps) * g + b


def _headln(x, S, B, eps=1e-5):
    mu = (x @ S) @ B
    xc = x - mu
    var = ((xc * xc) @ S) @ B
    return xc / jnp.sqrt(var + eps)


def _lrelu(x):
    return jnp.where(x >= 0, x, 0.01 * x)


def _proj_kernel(gf, sf, rf,
                 Wg, bg, gg, gb,
                 Ws, bs, sg, sb,
                 Wr, br, rg, rb,
                 Wqg, bqg, Wqs, bqs,
                 Wk, bk, Wv, bv,
                 Wt, bt, tg, tb,
                 S, B,
                 geo_p_o, sem_p_o, qg_o, qs_o, kg_o, ks_o, vg_o, vs_o, rsa_o):
    Sm, Bm = S[...], B[...]
    geo_p = _ln(gf[...] @ Wg[...] + bg[...], gg[...], gb[...])
    sem_p = _ln(sf[...] @ Ws[...] + bs[...], sg[...], sb[...])
    rsa_p = _ln(rf[...] @ Wr[...] + br[...], rg[...], rb[...])
    geo_p_o[...] = geo_p
    sem_p_o[...] = sem_p
    qg_o[...] = geo_p @ Wqg[...] + bqg[...]
    qs_o[...] = sem_p @ Wqs[...] + bqs[...]
    kg_o[...] = _headln(geo_p @ Wk[...] + bk[...], Sm, Bm)
    ks_o[...] = _headln(sem_p @ Wk[...] + bk[...], Sm, Bm)
    vg_o[...] = _headln(geo_p @ Wv[...] + bv[...], Sm, Bm)
    vs_o[...] = _headln(sem_p @ Wv[...] + bv[...], Sm, Bm)
    rsa_o[...] = _lrelu(_ln(rsa_p @ Wt[...] + bt[...], tg[...], tb[...]))


def _knn_kernel(pos_b, posT, m8_o, m16_o):
    pb = pos_b[...]
    pT = posT[...]
    d2 = jnp.zeros((BQ, L), jnp.float32)
    for c in range(3):
        diff = pb[:, c:c + 1] - pT[c:c + 1, :]
        d2 = d2 + diff * diff
    iota = jax.lax.broadcasted_iota(jnp.int32, (BQ, L), 1)
    sel = jnp.zeros((BQ, L), jnp.float32)
    cur = d2
    for t in range(16):
        v = jnp.min(cur, axis=1, keepdims=True)
        cand = jnp.where(cur == v, iota, L)
        j = jnp.min(cand, axis=1, keepdims=True)
        pick = iota == j
        sel = sel + pick.astype(jnp.float32)
        cur = jnp.where(pick, jnp.inf, cur)
        if t == 7:
            m8_o[...] = sel.astype(jnp.bfloat16)
    m16_o[...] = sel.astype(jnp.bfloat16)


def _attn_kernel(qp, kT, vT, m8, m16, o8_o, o16_o):
    scale = jnp.float32(1.0 / np.sqrt(DH))
    sidx = pl.program_id(1)
    cidx = 1 - sidx
    q = qp[0]
    m8f = m8[...].astype(jnp.float32)
    m16f = m16[...].astype(jnp.float32)
    bf = jnp.bfloat16
    ct0 = (((1,), (0,)), ((), ()))
    ct1 = (((1,), (1,)), ((), ()))
    outs8 = []
    outs16 = []
    for h in range(H):
        sl = slice(h * DH, (h + 1) * DH)
        qh = (q[:, sl] * scale).astype(bf)
        s_self = jax.lax.dot_general(qh, kT[sidx, sl, :], ct0,
                                     preferred_element_type=jnp.float32)
        s_cross = jax.lax.dot_general(qh, kT[cidx, sl, :], ct0,
                                      preferred_element_type=jnp.float32)
        m = jnp.maximum(jnp.max(s_self, axis=1, keepdims=True),
                        jnp.max(s_cross, axis=1, keepdims=True))
        e_self = jnp.exp(s_self - m)
        e_cross = jnp.exp(s_cross - m)
        zc = jnp.sum(e_cross, axis=1, keepdims=True)
        uc = jax.lax.dot_general(e_cross.astype(bf), vT[cidx, h], ct1,
                                 preferred_element_type=jnp.float32)
        e8 = e_self * m8f
        e16 = e_self * m16f
        z8 = zc + jnp.sum(e8, axis=1, keepdims=True)
        z16 = zc + jnp.sum(e16, axis=1, keepdims=True)
        vs_h = vT[sidx, h]
        u8 = uc + jax.lax.dot_general(e8.astype(bf), vs_h, ct1,
                                      preferred_element_type=jnp.float32)
        u16 = uc + jax.lax.dot_general(e16.astype(bf), vs_h, ct1,
                                       preferred_element_type=jnp.float32)
        outs8.append(u8 / z8)
        outs16.append(u16 / z16)
    o8_o[0] = jnp.concatenate(outs8, axis=1)
    o16_o[0] = jnp.concatenate(outs16, axis=1)


def _mix_kernel(g8, g16, s8, s16, geo_p, sem_p, rsa_out,
                Wo, bo, ln1g, ln1b, ln2g, ln2b,
                Wf1, bf1, f1g, f1b, Wf2, bf2, f2g, f2b,
                mix, out_o):
    mv = mix[...]
    swg = mv[0:1, 0:2]
    sws = mv[0:1, 2:4]
    wg = jnp.exp(swg - jnp.max(swg))
    wg = wg / jnp.sum(wg)
    ws = jnp.exp(sws - jnp.max(sws))
    ws = ws / jnp.sum(ws)
    wg0, wg1 = wg[0:1, 0:1], wg[0:1, 1:2]
    ws0, ws1 = ws[0:1, 0:1], ws[0:1, 1:2]
    alpha_g = mv[0:1, 4:5]
    beta_g = mv[0:1, 5:6]
    alpha_s = mv[0:1, 6:7]
    beta_s = mv[0:1, 7:8]

    Wo_m = Wo[...]
    bo_m = bo[...]
    geo_attn = (wg0 * g8[...] + wg1 * g16[...]) @ Wo_m + bo_m
    sem_attn = (ws0 * s8[...] + ws1 * s16[...]) @ Wo_m + bo_m
    geo_out = _ln(alpha_g * geo_p[...] + beta_g * geo_attn, ln1g[...], ln1b[...])
    sem_out = _ln(alpha_s * sem_p[...] + beta_s * sem_attn, ln2g[...], ln2b[...])
    W1 = Wf1[...]
    h1 = (geo_out @ W1[0:D, :] + sem_out @ W1[D:2 * D, :]
          + rsa_out[...] @ W1[2 * D:3 * D, :] + bf1[...])
    x = _lrelu(_ln(h1, f1g[...], f1b[...]))
    x = _lrelu(_ln(x @ Wf2[...] + bf2[...], f2g[...], f2b[...]))
    out_o[...] = x


def _row(v):
    return v.reshape(1, -1)


def _full_spec(shape):
    n = len(shape)
    return pl.BlockSpec(shape, lambda *_, _n=n: (0,) * _n)


def _blk_spec(cols):
    return pl.BlockSpec((BQ, cols), lambda i: (i, 0))


@jax.jit
def kernel(geo_feat, sem_feat, rsa_feat, pos, params):
    p = params
    f32 = jnp.float32

    posT = jnp.zeros((8, L), f32).at[0:3, :].set(pos.T)
    pos_pad = jnp.zeros((L, 8), f32).at[:, 0:3].set(pos)

    S = np.zeros((D, 128), np.float32)
    B = np.zeros((128, D), np.float32)
    for h in range(H):
        S[h * DH:(h + 1) * DH, h] = 1.0 / DH
        B[h, h * DH:(h + 1) * DH] = 1.0
    S = jnp.asarray(S)
    B = jnp.asarray(B)

    mix = jnp.zeros((1, 128), f32)
    mix = mix.at[0, 0:2].set(p['sw_g'])
    mix = mix.at[0, 2:4].set(p['sw_s'])
    mix = mix.at[0, 4].set(p['alpha_g'])
    mix = mix.at[0, 5].set(p['beta_g'])
    mix = mix.at[0, 6].set(p['alpha_s'])
    mix = mix.at[0, 7].set(p['beta_s'])

    LD = jax.ShapeDtypeStruct((L, D), f32)

    proj_in = [geo_feat, sem_feat, rsa_feat,
               p['Wg'], _row(p['bg']), _row(p['g_g']), _row(p['g_b']),
               p['Ws'], _row(p['bs']), _row(p['s_g']), _row(p['s_b']),
               p['Wr'], _row(p['br']), _row(p['r_g']), _row(p['r_b']),
               p['Wqg'], _row(p['bqg']), p['Wqs'], _row(p['bqs']),
               p['Wk'], _row(p['bk']), p['Wv'], _row(p['bv']),
               p['Wt'], _row(p['bt']), _row(p['t_g']), _row(p['t_b']),
               S, B]
    proj_specs = ([_blk_spec(GEO_DIM), _blk_spec(SEM_DIM), _blk_spec(RSA_DIM)]
                  + [_full_spec(a.shape) for a in proj_in[3:]])
    geo_p, sem_p, qg, qs, kg, ks, vg, vs, rsa_out = pl.pallas_call(
        _proj_kernel,
        grid=(NBLK,),
        in_specs=proj_specs,
        out_specs=[_blk_spec(D)] * 9,
        out_shape=[LD] * 9,
    )(*proj_in)

    m8, m16 = pl.pallas_call(
        _knn_kernel,
        grid=(NBLK,),
        in_specs=[_blk_spec(8), _full_spec((8, L))],
        out_specs=[_blk_spec(L)] * 2,
        out_shape=[jax.ShapeDtypeStruct((L, L), jnp.bfloat16)] * 2,
    )(pos_pad, posT)

    bfl = jnp.bfloat16
    Qp = jnp.stack([qg, qs])
    KT = jnp.stack([kg.T, ks.T]).astype(bfl)
    VT = jnp.stack([vg.T.reshape(H, DH, L),
                    vs.T.reshape(H, DH, L)]).astype(bfl)

    qspec = pl.BlockSpec((1, BQ, D), lambda qi, s: (s, qi, 0))
    mspec = pl.BlockSpec((BQ, L), lambda qi, s: (qi, 0))
    o8p, o16p = pl.pallas_call(
        _attn_kernel,
        grid=(NBLK, 2),
        in_specs=[qspec, _full_spec((2, D, L)), _full_spec((2, H, DH, L)),
                  mspec, mspec],
        out_specs=[qspec, qspec],
        out_shape=[jax.ShapeDtypeStruct((2, L, D), f32)] * 2,
    )(Qp, KT, VT, m8, m16)

    g8, s8 = o8p[0], o8p[1]
    g16, s16 = o16p[0], o16p[1]

    mix_in = [g8, g16, s8, s16, geo_p, sem_p, rsa_out,
              p['Wo'], _row(p['bo']),
              _row(p['ln1_g']), _row(p['ln1_b']),
              _row(p['ln2_g']), _row(p['ln2_b']),
              p['Wf1'], _row(p['bf1']), _row(p['f1_g']), _row(p['f1_b']),
              p['Wf2'], _row(p['bf2']), _row(p['f2_g']), _row(p['f2_b']),
              mix]
    mix_specs = ([_blk_spec(D)] * 7
                 + [_full_spec(a.shape) for a in mix_in[7:]])
    out = pl.pallas_call(
        _mix_kernel,
        grid=(NBLK,),
        in_specs=mix_specs,
        out_specs=_blk_spec(D),
        out_shape=LD,
    )(*mix_in)
    return out


# speedup vs baseline: 1.6748x; 1.2447x over previous
import functools

import jax
import jax.numpy as jnp
import numpy as np
from jax.experimental import pallas as pl
from jax.experimental.pallas import tpu as pltpu

L = 2048
GEO_DIM = 1536
SEM_DIM = 512
RSA_DIM = 64
D = 256
H = 8
DH = D // H
BQ = 256
NBLK = L // BQ


def _ln(x, g, b, eps=1e-5):
    mu = jnp.mean(x, axis=-1, keepdims=True)
    xc = x - mu
    var = jnp.mean(xc * xc, axis=-1, keepdims=True)
    return xc / jnp.sqrt(var + eps) * g + b


def _headln(x, S, B, eps=1e-5):
    mu = (x @ S) @ B
    xc = x - mu
    var = ((xc * xc) @ S) @ B
    return xc / jnp.sqrt(var + eps)


def _lrelu(x):
    return jnp.where(x >= 0, x, 0.01 * x)


def _proj_kernel(gf, sf, rf,
                 Wg, bg, gg, gb,
                 Ws, bs, sg, sb,
                 Wr, br, rg, rb,
                 Wqg, bqg, Wqs, bqs,
                 Wk, bk, Wv, bv,
                 Wt, bt, tg, tb,
                 S, B,
                 geo_p_o, sem_p_o, qg_o, qs_o, kg_o, ks_o, vg_o, vs_o, rsa_o):
    Sm, Bm = S[...], B[...]
    geo_p = _ln(gf[...] @ Wg[...] + bg[...], gg[...], gb[...])
    sem_p = _ln(sf[...] @ Ws[...] + bs[...], sg[...], sb[...])
    rsa_p = _ln(rf[...] @ Wr[...] + br[...], rg[...], rb[...])
    geo_p_o[...] = geo_p
    sem_p_o[...] = sem_p
    qg_o[...] = geo_p @ Wqg[...] + bqg[...]
    qs_o[...] = sem_p @ Wqs[...] + bqs[...]
    kg_o[...] = _headln(geo_p @ Wk[...] + bk[...], Sm, Bm)
    ks_o[...] = _headln(sem_p @ Wk[...] + bk[...], Sm, Bm)
    vg_o[...] = _headln(geo_p @ Wv[...] + bv[...], Sm, Bm)
    vs_o[...] = _headln(sem_p @ Wv[...] + bv[...], Sm, Bm)
    rsa_o[...] = _lrelu(_ln(rsa_p @ Wt[...] + bt[...], tg[...], tb[...]))


def _knn_kernel(pos_b, posT, m8_o, m16_o):
    pb = pos_b[...]
    pT = posT[...]
    d2 = jnp.zeros((BQ, L), jnp.float32)
    for c in range(3):
        diff = pb[:, c:c + 1] - pT[c:c + 1, :]
        d2 = d2 + diff * diff
    iota = jax.lax.broadcasted_iota(jnp.int32, (BQ, L), 1)
    sel = jnp.zeros((BQ, L), jnp.float32)
    cur = d2
    for t in range(16):
        v = jnp.min(cur, axis=1, keepdims=True)
        cand = jnp.where(cur == v, iota, L)
        j = jnp.min(cand, axis=1, keepdims=True)
        pick = iota == j
        sel = sel + pick.astype(jnp.float32)
        cur = jnp.where(pick, jnp.inf, cur)
        if t == 7:
            m8_o[...] = sel.astype(jnp.bfloat16)
    m16_o[...] = sel.astype(jnp.bfloat16)


def _attn_kernel(qp, kT, vT, m8, m16, o8_o, o16_o):
    scale = jnp.float32(1.0 / np.sqrt(DH))
    sidx = pl.program_id(1)
    cidx = 1 - sidx
    q = qp[0]
    m8f = m8[...].astype(jnp.float32)
    m16f = m16[...].astype(jnp.float32)
    bf = jnp.bfloat16
    ct0 = (((1,), (0,)), ((), ()))
    ct1 = (((1,), (1,)), ((), ()))
    outs8 = []
    outs16 = []
    for h in range(H):
        sl = slice(h * DH, (h + 1) * DH)
        qh = (q[:, sl] * scale).astype(bf)
        s_self = jax.lax.dot_general(qh, kT[sidx, sl, :], ct0,
                                     preferred_element_type=jnp.float32)
        s_cross = jax.lax.dot_general(qh, kT[cidx, sl, :], ct0,
                                      preferred_element_type=jnp.float32)
        e_self = jnp.exp(s_self)
        e_cross = jnp.exp(s_cross)
        uz_c = jax.lax.dot_general(e_cross.astype(bf), vT[cidx, h], ct1,
                                   preferred_element_type=jnp.float32)
        e8 = e_self * m8f
        e16 = e_self * m16f
        vs_h = vT[sidx, h]
        uz8 = uz_c + jax.lax.dot_general(e8.astype(bf), vs_h, ct1,
                                         preferred_element_type=jnp.float32)
        uz16 = uz_c + jax.lax.dot_general(e16.astype(bf), vs_h, ct1,
                                          preferred_element_type=jnp.float32)
        outs8.append(uz8[:, 0:DH] / uz8[:, DH:DH + 1])
        outs16.append(uz16[:, 0:DH] / uz16[:, DH:DH + 1])
    o8_o[0] = jnp.concatenate(outs8, axis=1)
    o16_o[0] = jnp.concatenate(outs16, axis=1)


def _mix_kernel(g8, g16, s8, s16, geo_p, sem_p, rsa_out,
                Wo, bo, ln1g, ln1b, ln2g, ln2b,
                Wf1, bf1, f1g, f1b, Wf2, bf2, f2g, f2b,
                mix, out_o):
    mv = mix[...]
    swg = mv[0:1, 0:2]
    sws = mv[0:1, 2:4]
    wg = jnp.exp(swg - jnp.max(swg))
    wg = wg / jnp.sum(wg)
    ws = jnp.exp(sws - jnp.max(sws))
    ws = ws / jnp.sum(ws)
    wg0, wg1 = wg[0:1, 0:1], wg[0:1, 1:2]
    ws0, ws1 = ws[0:1, 0:1], ws[0:1, 1:2]
    alpha_g = mv[0:1, 4:5]
    beta_g = mv[0:1, 5:6]
    alpha_s = mv[0:1, 6:7]
    beta_s = mv[0:1, 7:8]

    Wo_m = Wo[...]
    bo_m = bo[...]
    geo_attn = (wg0 * g8[...] + wg1 * g16[...]) @ Wo_m + bo_m
    sem_attn = (ws0 * s8[...] + ws1 * s16[...]) @ Wo_m + bo_m
    geo_out = _ln(alpha_g * geo_p[...] + beta_g * geo_attn, ln1g[...], ln1b[...])
    sem_out = _ln(alpha_s * sem_p[...] + beta_s * sem_attn, ln2g[...], ln2b[...])
    W1 = Wf1[...]
    h1 = (geo_out @ W1[0:D, :] + sem_out @ W1[D:2 * D, :]
          + rsa_out[...] @ W1[2 * D:3 * D, :] + bf1[...])
    x = _lrelu(_ln(h1, f1g[...], f1b[...]))
    x = _lrelu(_ln(x @ Wf2[...] + bf2[...], f2g[...], f2b[...]))
    out_o[...] = x


def _row(v):
    return v.reshape(1, -1)


def _full_spec(shape):
    n = len(shape)
    return pl.BlockSpec(shape, lambda *_, _n=n: (0,) * _n)


def _blk_spec(cols):
    return pl.BlockSpec((BQ, cols), lambda i: (i, 0))


@jax.jit
def kernel(geo_feat, sem_feat, rsa_feat, pos, params):
    p = params
    f32 = jnp.float32

    posT = jnp.zeros((8, L), f32).at[0:3, :].set(pos.T)
    pos_pad = jnp.zeros((L, 8), f32).at[:, 0:3].set(pos)

    S = np.zeros((D, 128), np.float32)
    B = np.zeros((128, D), np.float32)
    for h in range(H):
        S[h * DH:(h + 1) * DH, h] = 1.0 / DH
        B[h, h * DH:(h + 1) * DH] = 1.0
    S = jnp.asarray(S)
    B = jnp.asarray(B)

    mix = jnp.zeros((1, 128), f32)
    mix = mix.at[0, 0:2].set(p['sw_g'])
    mix = mix.at[0, 2:4].set(p['sw_s'])
    mix = mix.at[0, 4].set(p['alpha_g'])
    mix = mix.at[0, 5].set(p['beta_g'])
    mix = mix.at[0, 6].set(p['alpha_s'])
    mix = mix.at[0, 7].set(p['beta_s'])

    LD = jax.ShapeDtypeStruct((L, D), f32)

    proj_in = [geo_feat, sem_feat, rsa_feat,
               p['Wg'], _row(p['bg']), _row(p['g_g']), _row(p['g_b']),
               p['Ws'], _row(p['bs']), _row(p['s_g']), _row(p['s_b']),
               p['Wr'], _row(p['br']), _row(p['r_g']), _row(p['r_b']),
               p['Wqg'], _row(p['bqg']), p['Wqs'], _row(p['bqs']),
               p['Wk'], _row(p['bk']), p['Wv'], _row(p['bv']),
               p['Wt'], _row(p['bt']), _row(p['t_g']), _row(p['t_b']),
               S, B]
    proj_specs = ([_blk_spec(GEO_DIM), _blk_spec(SEM_DIM), _blk_spec(RSA_DIM)]
                  + [_full_spec(a.shape) for a in proj_in[3:]])
    geo_p, sem_p, qg, qs, kg, ks, vg, vs, rsa_out = pl.pallas_call(
        _proj_kernel,
        grid=(NBLK,),
        in_specs=proj_specs,
        out_specs=[_blk_spec(D)] * 9,
        out_shape=[LD] * 9,
    )(*proj_in)

    m8, m16 = pl.pallas_call(
        _knn_kernel,
        grid=(NBLK,),
        in_specs=[_blk_spec(8), _full_spec((8, L))],
        out_specs=[_blk_spec(L)] * 2,
        out_shape=[jax.ShapeDtypeStruct((L, L), jnp.bfloat16)] * 2,
    )(pos_pad, posT)

    bfl = jnp.bfloat16
    Qp = jnp.stack([qg, qs])
    KT = jnp.stack([kg.T, ks.T]).astype(bfl)
    VT = jnp.stack([vg.T.reshape(H, DH, L),
                    vs.T.reshape(H, DH, L)]).astype(bfl)
    VT = jnp.concatenate(
        [VT, jnp.ones((2, H, 1, L), bfl), jnp.zeros((2, H, 7, L), bfl)],
        axis=2)

    qspec = pl.BlockSpec((1, BQ, D), lambda qi, s: (s, qi, 0))
    mspec = pl.BlockSpec((BQ, L), lambda qi, s: (qi, 0))
    o8p, o16p = pl.pallas_call(
        _attn_kernel,
        grid=(NBLK, 2),
        in_specs=[qspec, _full_spec((2, D, L)), _full_spec((2, H, DH + 8, L)),
                  mspec, mspec],
        out_specs=[qspec, qspec],
        out_shape=[jax.ShapeDtypeStruct((2, L, D), f32)] * 2,
    )(Qp, KT, VT, m8, m16)

    g8, s8 = o8p[0], o8p[1]
    g16, s16 = o16p[0], o16p[1]

    mix_in = [g8, g16, s8, s16, geo_p, sem_p, rsa_out,
              p['Wo'], _row(p['bo']),
              _row(p['ln1_g']), _row(p['ln1_b']),
              _row(p['ln2_g']), _row(p['ln2_b']),
              p['Wf1'], _row(p['bf1']), _row(p['f1_g']), _row(p['f1_b']),
              p['Wf2'], _row(p['bf2']), _row(p['f2_g']), _row(p['f2_b']),
              mix]
    mix_specs = ([_blk_spec(D)] * 7
                 + [_full_spec(a.shape) for a in mix_in[7:]])
    out = pl.pallas_call(
        _mix_kernel,
        grid=(NBLK,),
        in_specs=mix_specs,
        out_specs=_blk_spec(D),
        out_shape=LD,
    )(*mix_in)
    return out


# speedup vs baseline: 1.8072x; 1.0790x over previous
import functools

import jax
import jax.numpy as jnp
import numpy as np
from jax.experimental import pallas as pl
from jax.experimental.pallas import tpu as pltpu

L = 2048
GEO_DIM = 1536
SEM_DIM = 512
RSA_DIM = 64
D = 256
H = 8
DH = D // H
BQ = 256
NBLK = L // BQ


def _ln(x, g, b, eps=1e-5):
    mu = jnp.mean(x, axis=-1, keepdims=True)
    xc = x - mu
    var = jnp.mean(xc * xc, axis=-1, keepdims=True)
    return xc / jnp.sqrt(var + eps) * g + b


def _headln(x, S, B, eps=1e-5):
    mu = (x @ S) @ B
    xc = x - mu
    var = ((xc * xc) @ S) @ B
    return xc / jnp.sqrt(var + eps)


def _lrelu(x):
    return jnp.where(x >= 0, x, 0.01 * x)


def _proj_kernel(gf, sf, rf,
                 Wg, bg, gg, gb,
                 Ws, bs, sg, sb,
                 Wr, br, rg, rb,
                 Wqg, bqg, Wqs, bqs,
                 Wk, bk, Wv, bv,
                 Wt, bt, tg, tb,
                 S, B,
                 geo_p_o, sem_p_o, qp_o, kT_o, vT_o, rsa_o):
    bf = jnp.bfloat16
    Sm, Bm = S[...], B[...]
    geo_p = _ln(gf[...] @ Wg[...] + bg[...], gg[...], gb[...])
    sem_p = _ln(sf[...] @ Ws[...] + bs[...], sg[...], sb[...])
    rsa_p = _ln(rf[...] @ Wr[...] + br[...], rg[...], rb[...])
    geo_p_o[...] = geo_p
    sem_p_o[...] = sem_p
    qp_o[0] = geo_p @ Wqg[...] + bqg[...]
    qp_o[1] = sem_p @ Wqs[...] + bqs[...]
    kT_o[0] = _headln(geo_p @ Wk[...] + bk[...], Sm, Bm).T.astype(bf)
    kT_o[1] = _headln(sem_p @ Wk[...] + bk[...], Sm, Bm).T.astype(bf)
    one = jnp.ones((H, 1, BQ), bf)
    zero = jnp.zeros((H, 7, BQ), bf)
    for side, p in ((0, geo_p), (1, sem_p)):
        v = _headln(p @ Wv[...] + bv[...], Sm, Bm).T.astype(bf)
        vt = v.reshape(H, DH, BQ)
        vT_o[side] = jnp.concatenate([vt, one, zero], axis=1)
    rsa_o[...] = _lrelu(_ln(rsa_p @ Wt[...] + bt[...], tg[...], tb[...]))


def _knn_kernel(pos_b, posT, m8_o, m16_o):
    pb = pos_b[...]
    pT = posT[...]
    d2 = jnp.zeros((BQ, L), jnp.float32)
    for c in range(3):
        diff = pb[:, c:c + 1] - pT[c:c + 1, :]
        d2 = d2 + diff * diff
    iota = jax.lax.broadcasted_iota(jnp.int32, (BQ, L), 1)
    sel = jnp.zeros((BQ, L), jnp.float32)
    cur = d2
    for t in range(16):
        v = jnp.min(cur, axis=1, keepdims=True)
        cand = jnp.where(cur == v, iota, L)
        j = jnp.min(cand, axis=1, keepdims=True)
        pick = iota == j
        sel = sel + pick.astype(jnp.float32)
        cur = jnp.where(pick, jnp.inf, cur)
        if t == 7:
            m8_o[...] = sel.astype(jnp.bfloat16)
    m16_o[...] = sel.astype(jnp.bfloat16)


def _attn_kernel(qp, kT, vT, m8, m16, o8_o, o16_o):
    scale = jnp.float32(1.0 / np.sqrt(DH))
    sidx = pl.program_id(1)
    cidx = 1 - sidx
    q = qp[0]
    m8f = m8[...]
    m16f = m16[...]
    bf = jnp.bfloat16
    ct0 = (((1,), (0,)), ((), ()))
    ct1 = (((1,), (1,)), ((), ()))
    outs8 = []
    outs16 = []
    for h in range(H):
        sl = slice(h * DH, (h + 1) * DH)
        qh = (q[:, sl] * scale).astype(bf)
        s_self = jax.lax.dot_general(qh, kT[sidx, sl, :], ct0,
                                     preferred_element_type=jnp.float32)
        s_cross = jax.lax.dot_general(qh, kT[cidx, sl, :], ct0,
                                      preferred_element_type=jnp.float32)
        e_self = jnp.exp(s_self).astype(bf)
        e_cross = jnp.exp(s_cross).astype(bf)
        uz_c = jax.lax.dot_general(e_cross, vT[cidx, h], ct1,
                                   preferred_element_type=jnp.float32)
        e8 = e_self * m8f
        e16 = e_self * m16f
        vs_h = vT[sidx, h]
        uz8 = uz_c + jax.lax.dot_general(e8, vs_h, ct1,
                                         preferred_element_type=jnp.float32)
        uz16 = uz_c + jax.lax.dot_general(e16, vs_h, ct1,
                                          preferred_element_type=jnp.float32)
        outs8.append(uz8[:, 0:DH] / uz8[:, DH:DH + 1])
        outs16.append(uz16[:, 0:DH] / uz16[:, DH:DH + 1])
    o8_o[0] = jnp.concatenate(outs8, axis=1)
    o16_o[0] = jnp.concatenate(outs16, axis=1)


def _mix_kernel(g8, g16, s8, s16, geo_p, sem_p, rsa_out,
                Wo, bo, ln1g, ln1b, ln2g, ln2b,
                Wf1, bf1, f1g, f1b, Wf2, bf2, f2g, f2b,
                mix, out_o):
    mv = mix[...]
    swg = mv[0:1, 0:2]
    sws = mv[0:1, 2:4]
    wg = jnp.exp(swg - jnp.max(swg))
    wg = wg / jnp.sum(wg)
    ws = jnp.exp(sws - jnp.max(sws))
    ws = ws / jnp.sum(ws)
    wg0, wg1 = wg[0:1, 0:1], wg[0:1, 1:2]
    ws0, ws1 = ws[0:1, 0:1], ws[0:1, 1:2]
    alpha_g = mv[0:1, 4:5]
    beta_g = mv[0:1, 5:6]
    alpha_s = mv[0:1, 6:7]
    beta_s = mv[0:1, 7:8]

    Wo_m = Wo[...]
    bo_m = bo[...]
    geo_attn = (wg0 * g8[0] + wg1 * g16[0]) @ Wo_m + bo_m
    sem_attn = (ws0 * s8[0] + ws1 * s16[0]) @ Wo_m + bo_m
    geo_out = _ln(alpha_g * geo_p[...] + beta_g * geo_attn, ln1g[...], ln1b[...])
    sem_out = _ln(alpha_s * sem_p[...] + beta_s * sem_attn, ln2g[...], ln2b[...])
    W1 = Wf1[...]
    h1 = (geo_out @ W1[0:D, :] + sem_out @ W1[D:2 * D, :]
          + rsa_out[...] @ W1[2 * D:3 * D, :] + bf1[...])
    x = _lrelu(_ln(h1, f1g[...], f1b[...]))
    x = _lrelu(_ln(x @ Wf2[...] + bf2[...], f2g[...], f2b[...]))
    out_o[...] = x


def _row(v):
    return v.reshape(1, -1)


def _full_spec(shape):
    n = len(shape)
    return pl.BlockSpec(shape, lambda *_, _n=n: (0,) * _n)


def _blk_spec(cols):
    return pl.BlockSpec((BQ, cols), lambda i: (i, 0))


@jax.jit
def kernel(geo_feat, sem_feat, rsa_feat, pos, params):
    p = params
    f32 = jnp.float32

    posT = jnp.zeros((8, L), f32).at[0:3, :].set(pos.T)
    pos_pad = jnp.zeros((L, 8), f32).at[:, 0:3].set(pos)

    S = np.zeros((D, 128), np.float32)
    B = np.zeros((128, D), np.float32)
    for h in range(H):
        S[h * DH:(h + 1) * DH, h] = 1.0 / DH
        B[h, h * DH:(h + 1) * DH] = 1.0
    S = jnp.asarray(S)
    B = jnp.asarray(B)

    mix = jnp.zeros((1, 128), f32)
    mix = mix.at[0, 0:2].set(p['sw_g'])
    mix = mix.at[0, 2:4].set(p['sw_s'])
    mix = mix.at[0, 4].set(p['alpha_g'])
    mix = mix.at[0, 5].set(p['beta_g'])
    mix = mix.at[0, 6].set(p['alpha_s'])
    mix = mix.at[0, 7].set(p['beta_s'])

    LD = jax.ShapeDtypeStruct((L, D), f32)

    proj_in = [geo_feat, sem_feat, rsa_feat,
               p['Wg'], _row(p['bg']), _row(p['g_g']), _row(p['g_b']),
               p['Ws'], _row(p['bs']), _row(p['s_g']), _row(p['s_b']),
               p['Wr'], _row(p['br']), _row(p['r_g']), _row(p['r_b']),
               p['Wqg'], _row(p['bqg']), p['Wqs'], _row(p['bqs']),
               p['Wk'], _row(p['bk']), p['Wv'], _row(p['bv']),
               p['Wt'], _row(p['bt']), _row(p['t_g']), _row(p['t_b']),
               S, B]
    proj_specs = ([_blk_spec(GEO_DIM), _blk_spec(SEM_DIM), _blk_spec(RSA_DIM)]
                  + [_full_spec(a.shape) for a in proj_in[3:]])
    bfl = jnp.bfloat16
    geo_p, sem_p, Qp, KT, VT, rsa_out = pl.pallas_call(
        _proj_kernel,
        grid=(NBLK,),
        in_specs=proj_specs,
        out_specs=[_blk_spec(D), _blk_spec(D),
                   pl.BlockSpec((2, BQ, D), lambda i: (0, i, 0)),
                   pl.BlockSpec((2, D, BQ), lambda i: (0, 0, i)),
                   pl.BlockSpec((2, H, DH + 8, BQ), lambda i: (0, 0, 0, i)),
                   _blk_spec(D)],
        out_shape=[LD, LD,
                   jax.ShapeDtypeStruct((2, L, D), f32),
                   jax.ShapeDtypeStruct((2, D, L), bfl),
                   jax.ShapeDtypeStruct((2, H, DH + 8, L), bfl),
                   LD],
    )(*proj_in)

    m8, m16 = pl.pallas_call(
        _knn_kernel,
        grid=(NBLK,),
        in_specs=[_blk_spec(8), _full_spec((8, L))],
        out_specs=[_blk_spec(L)] * 2,
        out_shape=[jax.ShapeDtypeStruct((L, L), jnp.bfloat16)] * 2,
    )(pos_pad, posT)

    qspec = pl.BlockSpec((1, BQ, D), lambda qi, s: (s, qi, 0))
    mspec = pl.BlockSpec((BQ, L), lambda qi, s: (qi, 0))
    o8p, o16p = pl.pallas_call(
        _attn_kernel,
        grid=(NBLK, 2),
        in_specs=[qspec, _full_spec((2, D, L)), _full_spec((2, H, DH + 8, L)),
                  mspec, mspec],
        out_specs=[qspec, qspec],
        out_shape=[jax.ShapeDtypeStruct((2, L, D), f32)] * 2,
    )(Qp, KT, VT, m8, m16)

    gsp = pl.BlockSpec((1, BQ, D), lambda i: (0, i, 0))
    ssp = pl.BlockSpec((1, BQ, D), lambda i: (1, i, 0))
    mix_in = [o8p, o16p, o8p, o16p, geo_p, sem_p, rsa_out,
              p['Wo'], _row(p['bo']),
              _row(p['ln1_g']), _row(p['ln1_b']),
              _row(p['ln2_g']), _row(p['ln2_b']),
              p['Wf1'], _row(p['bf1']), _row(p['f1_g']), _row(p['f1_b']),
              p['Wf2'], _row(p['bf2']), _row(p['f2_g']), _row(p['f2_b']),
              mix]
    mix_specs = ([gsp, gsp, ssp, ssp] + [_blk_spec(D)] * 3
                 + [_full_spec(a.shape) for a in mix_in[7:]])
    out = pl.pallas_call(
        _mix_kernel,
        grid=(NBLK,),
        in_specs=mix_specs,
        out_specs=_blk_spec(D),
        out_shape=LD,
    )(*mix_in)
    return out


# speedup vs baseline: 1.9720x; 1.0912x over previous
import functools

import jax
import jax.numpy as jnp
import numpy as np
from jax.experimental import pallas as pl
from jax.experimental.pallas import tpu as pltpu

L = 2048
GEO_DIM = 1536
SEM_DIM = 512
RSA_DIM = 64
D = 256
H = 8
DH = D // H
BQ = 256
NBLK = L // BQ


def _ln(x, g, b, eps=1e-5):
    mu = jnp.mean(x, axis=-1, keepdims=True)
    xc = x - mu
    var = jnp.mean(xc * xc, axis=-1, keepdims=True)
    return xc / jnp.sqrt(var + eps) * g + b


def _headln(x, S, B, eps=1e-5):
    mu = (x @ S) @ B
    xc = x - mu
    var = ((xc * xc) @ S) @ B
    return xc / jnp.sqrt(var + eps)


def _lrelu(x):
    return jnp.where(x >= 0, x, 0.01 * x)


def _proj_kernel(gf, sf, rf,
                 Wg, bg, gg, gb,
                 Ws, bs, sg, sb,
                 Wr, br, rg, rb,
                 Wqg, bqg, Wqs, bqs,
                 Wk, bk, Wv, bv,
                 Wt, bt, tg, tb,
                 S, B,
                 geo_p_o, sem_p_o, qp_o, kT_o, vT_o, rsa_o):
    bf = jnp.bfloat16
    Sm, Bm = S[...], B[...]
    geo_p = _ln(gf[...] @ Wg[...] + bg[...], gg[...], gb[...])
    sem_p = _ln(sf[...] @ Ws[...] + bs[...], sg[...], sb[...])
    rsa_p = _ln(rf[...] @ Wr[...] + br[...], rg[...], rb[...])
    geo_p_o[...] = geo_p
    sem_p_o[...] = sem_p
    qp_o[0] = geo_p @ Wqg[...] + bqg[...]
    qp_o[1] = sem_p @ Wqs[...] + bqs[...]
    kT_o[0] = _headln(geo_p @ Wk[...] + bk[...], Sm, Bm).T.astype(bf)
    kT_o[1] = _headln(sem_p @ Wk[...] + bk[...], Sm, Bm).T.astype(bf)
    one = jnp.ones((H, 1, BQ), bf)
    zero = jnp.zeros((H, 7, BQ), bf)
    for side, p in ((0, geo_p), (1, sem_p)):
        v = _headln(p @ Wv[...] + bv[...], Sm, Bm).T.astype(bf)
        vt = v.reshape(H, DH, BQ)
        vT_o[side] = jnp.concatenate([vt, one, zero], axis=1)
    rsa_o[...] = _lrelu(_ln(rsa_p @ Wt[...] + bt[...], tg[...], tb[...]))


def _knn_kernel(pos_b, posT, m8_o, m16_o):
    pb = pos_b[...]
    pT = posT[...]
    d2 = jnp.zeros((BQ, L), jnp.float32)
    for c in range(3):
        diff = pb[:, c:c + 1] - pT[c:c + 1, :]
        d2 = d2 + diff * diff
    iota = jax.lax.broadcasted_iota(jnp.int32, (BQ, L), 1)
    key = jax.lax.bitcast_convert_type(d2, jnp.int32)
    key = (key & jnp.int32(-2048)) | iota
    sel = jnp.zeros((BQ, L), jnp.float32)
    for t in range(16):
        v = jnp.min(key, axis=1, keepdims=True)
        pick = key == v
        sel = jnp.where(pick, jnp.float32(1.0), sel)
        key = jnp.where(pick, jnp.int32(0x7FFFFFFF), key)
        if t == 7:
            m8_o[...] = sel.astype(jnp.bfloat16)
    m16_o[...] = sel.astype(jnp.bfloat16)


def _attn_kernel(qp, kT, vT, m8, m16, o8_o, o16_o):
    scale = jnp.float32(1.0 / np.sqrt(DH))
    sidx = pl.program_id(1)
    cidx = 1 - sidx
    q = qp[0]
    m8f = m8[...]
    m16f = m16[...]
    bf = jnp.bfloat16
    ct0 = (((1,), (0,)), ((), ()))
    ct1 = (((1,), (1,)), ((), ()))
    outs8 = []
    outs16 = []
    for h in range(H):
        sl = slice(h * DH, (h + 1) * DH)
        qh = (q[:, sl] * scale).astype(bf)
        s_self = jax.lax.dot_general(qh, kT[sidx, sl, :], ct0,
                                     preferred_element_type=jnp.float32)
        s_cross = jax.lax.dot_general(qh, kT[cidx, sl, :], ct0,
                                      preferred_element_type=jnp.float32)
        e_self = jnp.exp(s_self).astype(bf)
        e_cross = jnp.exp(s_cross).astype(bf)
        uz_c = jax.lax.dot_general(e_cross, vT[cidx, h], ct1,
                                   preferred_element_type=jnp.float32)
        e8 = e_self * m8f
        e16 = e_self * m16f
        vs_h = vT[sidx, h]
        uz8 = uz_c + jax.lax.dot_general(e8, vs_h, ct1,
                                         preferred_element_type=jnp.float32)
        uz16 = uz_c + jax.lax.dot_general(e16, vs_h, ct1,
                                          preferred_element_type=jnp.float32)
        outs8.append(uz8[:, 0:DH] / uz8[:, DH:DH + 1])
        outs16.append(uz16[:, 0:DH] / uz16[:, DH:DH + 1])
    o8_o[0] = jnp.concatenate(outs8, axis=1)
    o16_o[0] = jnp.concatenate(outs16, axis=1)


def _mix_kernel(g8, g16, s8, s16, geo_p, sem_p, rsa_out,
                Wo, bo, ln1g, ln1b, ln2g, ln2b,
                Wf1, bf1, f1g, f1b, Wf2, bf2, f2g, f2b,
                mix, out_o):
    mv = mix[...]
    swg = mv[0:1, 0:2]
    sws = mv[0:1, 2:4]
    wg = jnp.exp(swg - jnp.max(swg))
    wg = wg / jnp.sum(wg)
    ws = jnp.exp(sws - jnp.max(sws))
    ws = ws / jnp.sum(ws)
    wg0, wg1 = wg[0:1, 0:1], wg[0:1, 1:2]
    ws0, ws1 = ws[0:1, 0:1], ws[0:1, 1:2]
    alpha_g = mv[0:1, 4:5]
    beta_g = mv[0:1, 5:6]
    alpha_s = mv[0:1, 6:7]
    beta_s = mv[0:1, 7:8]

    Wo_m = Wo[...]
    bo_m = bo[...]
    geo_attn = (wg0 * g8[0] + wg1 * g16[0]) @ Wo_m + bo_m
    sem_attn = (ws0 * s8[0] + ws1 * s16[0]) @ Wo_m + bo_m
    geo_out = _ln(alpha_g * geo_p[...] + beta_g * geo_attn, ln1g[...], ln1b[...])
    sem_out = _ln(alpha_s * sem_p[...] + beta_s * sem_attn, ln2g[...], ln2b[...])
    W1 = Wf1[...]
    h1 = (geo_out @ W1[0:D, :] + sem_out @ W1[D:2 * D, :]
          + rsa_out[...] @ W1[2 * D:3 * D, :] + bf1[...])
    x = _lrelu(_ln(h1, f1g[...], f1b[...]))
    x = _lrelu(_ln(x @ Wf2[...] + bf2[...], f2g[...], f2b[...]))
    out_o[...] = x


def _row(v):
    return v.reshape(1, -1)


def _full_spec(shape):
    n = len(shape)
    return pl.BlockSpec(shape, lambda *_, _n=n: (0,) * _n)


def _blk_spec(cols):
    return pl.BlockSpec((BQ, cols), lambda i: (i, 0))


@jax.jit
def kernel(geo_feat, sem_feat, rsa_feat, pos, params):
    p = params
    f32 = jnp.float32

    posT = jnp.zeros((8, L), f32).at[0:3, :].set(pos.T)
    pos_pad = jnp.zeros((L, 8), f32).at[:, 0:3].set(pos)

    S = np.zeros((D, 128), np.float32)
    B = np.zeros((128, D), np.float32)
    for h in range(H):
        S[h * DH:(h + 1) * DH, h] = 1.0 / DH
        B[h, h * DH:(h + 1) * DH] = 1.0
    S = jnp.asarray(S)
    B = jnp.asarray(B)

    mix = jnp.zeros((1, 128), f32)
    mix = mix.at[0, 0:2].set(p['sw_g'])
    mix = mix.at[0, 2:4].set(p['sw_s'])
    mix = mix.at[0, 4].set(p['alpha_g'])
    mix = mix.at[0, 5].set(p['beta_g'])
    mix = mix.at[0, 6].set(p['alpha_s'])
    mix = mix.at[0, 7].set(p['beta_s'])

    LD = jax.ShapeDtypeStruct((L, D), f32)

    proj_in = [geo_feat, sem_feat, rsa_feat,
               p['Wg'], _row(p['bg']), _row(p['g_g']), _row(p['g_b']),
               p['Ws'], _row(p['bs']), _row(p['s_g']), _row(p['s_b']),
               p['Wr'], _row(p['br']), _row(p['r_g']), _row(p['r_b']),
               p['Wqg'], _row(p['bqg']), p['Wqs'], _row(p['bqs']),
               p['Wk'], _row(p['bk']), p['Wv'], _row(p['bv']),
               p['Wt'], _row(p['bt']), _row(p['t_g']), _row(p['t_b']),
               S, B]
    proj_specs = ([_blk_spec(GEO_DIM), _blk_spec(SEM_DIM), _blk_spec(RSA_DIM)]
                  + [_full_spec(a.shape) for a in proj_in[3:]])
    bfl = jnp.bfloat16
    geo_p, sem_p, Qp, KT, VT, rsa_out = pl.pallas_call(
        _proj_kernel,
        grid=(NBLK,),
        in_specs=proj_specs,
        out_specs=[_blk_spec(D), _blk_spec(D),
                   pl.BlockSpec((2, BQ, D), lambda i: (0, i, 0)),
                   pl.BlockSpec((2, D, BQ), lambda i: (0, 0, i)),
                   pl.BlockSpec((2, H, DH + 8, BQ), lambda i: (0, 0, 0, i)),
                   _blk_spec(D)],
        out_shape=[LD, LD,
                   jax.ShapeDtypeStruct((2, L, D), f32),
                   jax.ShapeDtypeStruct((2, D, L), bfl),
                   jax.ShapeDtypeStruct((2, H, DH + 8, L), bfl),
                   LD],
    )(*proj_in)

    m8, m16 = pl.pallas_call(
        _knn_kernel,
        grid=(NBLK,),
        in_specs=[_blk_spec(8), _full_spec((8, L))],
        out_specs=[_blk_spec(L)] * 2,
        out_shape=[jax.ShapeDtypeStruct((L, L), jnp.bfloat16)] * 2,
    )(pos_pad, posT)

    qspec = pl.BlockSpec((1, BQ, D), lambda qi, s: (s, qi, 0))
    mspec = pl.BlockSpec((BQ, L), lambda qi, s: (qi, 0))
    o8p, o16p = pl.pallas_call(
        _attn_kernel,
        grid=(NBLK, 2),
        in_specs=[qspec, _full_spec((2, D, L)), _full_spec((2, H, DH + 8, L)),
                  mspec, mspec],
        out_specs=[qspec, qspec],
        out_shape=[jax.ShapeDtypeStruct((2, L, D), f32)] * 2,
    )(Qp, KT, VT, m8, m16)

    gsp = pl.BlockSpec((1, BQ, D), lambda i: (0, i, 0))
    ssp = pl.BlockSpec((1, BQ, D), lambda i: (1, i, 0))
    mix_in = [o8p, o16p, o8p, o16p, geo_p, sem_p, rsa_out,
              p['Wo'], _row(p['bo']),
              _row(p['ln1_g']), _row(p['ln1_b']),
              _row(p['ln2_g']), _row(p['ln2_b']),
              p['Wf1'], _row(p['bf1']), _row(p['f1_g']), _row(p['f1_b']),
              p['Wf2'], _row(p['bf2']), _row(p['f2_g']), _row(p['f2_b']),
              mix]
    mix_specs = ([gsp, gsp, ssp, ssp] + [_blk_spec(D)] * 3
                 + [_full_spec(a.shape) for a in mix_in[7:]])
    out = pl.pallas_call(
        _mix_kernel,
        grid=(NBLK,),
        in_specs=mix_specs,
        out_specs=_blk_spec(D),
        out_shape=LD,
    )(*mix_in)
    return out


# speedup vs baseline: 2.0036x; 1.0160x over previous
import functools

import jax
import jax.numpy as jnp
import numpy as np
from jax.experimental import pallas as pl
from jax.experimental.pallas import tpu as pltpu

L = 2048
GEO_DIM = 1536
SEM_DIM = 512
RSA_DIM = 64
D = 256
H = 8
DH = D // H
BQ = 512
NBLK = L // BQ


def _ln(x, g, b, eps=1e-5):
    mu = jnp.mean(x, axis=-1, keepdims=True)
    xc = x - mu
    var = jnp.mean(xc * xc, axis=-1, keepdims=True)
    return xc / jnp.sqrt(var + eps) * g + b


def _headln(x, S, B, eps=1e-5):
    mu = (x @ S) @ B
    xc = x - mu
    var = ((xc * xc) @ S) @ B
    return xc / jnp.sqrt(var + eps)


def _lrelu(x):
    return jnp.where(x >= 0, x, 0.01 * x)


def _proj_kernel(gf, sf, rf,
                 Wg, bg, gg, gb,
                 Ws, bs, sg, sb,
                 Wr, br, rg, rb,
                 Wqg, bqg, Wqs, bqs,
                 Wk, bk, Wv, bv,
                 Wt, bt, tg, tb,
                 S, B,
                 geo_p_o, sem_p_o, qp_o, kT_o, vT_o, rsa_o):
    bf = jnp.bfloat16
    Sm, Bm = S[...], B[...]
    geo_p = _ln(gf[...] @ Wg[...] + bg[...], gg[...], gb[...])
    sem_p = _ln(sf[...] @ Ws[...] + bs[...], sg[...], sb[...])
    rsa_p = _ln(rf[...] @ Wr[...] + br[...], rg[...], rb[...])
    geo_p_o[...] = geo_p
    sem_p_o[...] = sem_p
    qp_o[0] = geo_p @ Wqg[...] + bqg[...]
    qp_o[1] = sem_p @ Wqs[...] + bqs[...]
    kT_o[0] = _headln(geo_p @ Wk[...] + bk[...], Sm, Bm).T.astype(bf)
    kT_o[1] = _headln(sem_p @ Wk[...] + bk[...], Sm, Bm).T.astype(bf)
    one = jnp.ones((H, 1, BQ), bf)
    zero = jnp.zeros((H, 7, BQ), bf)
    for side, p in ((0, geo_p), (1, sem_p)):
        v = _headln(p @ Wv[...] + bv[...], Sm, Bm).T.astype(bf)
        vt = v.reshape(H, DH, BQ)
        vT_o[side] = jnp.concatenate([vt, one, zero], axis=1)
    rsa_o[...] = _lrelu(_ln(rsa_p @ Wt[...] + bt[...], tg[...], tb[...]))


def _knn_kernel(pos_b, posT, m8_o, m16_o):
    pb = pos_b[...]
    pT = posT[...]
    d2 = jnp.zeros((BQ, L), jnp.float32)
    for c in range(3):
        diff = pb[:, c:c + 1] - pT[c:c + 1, :]
        d2 = d2 + diff * diff
    iota = jax.lax.broadcasted_iota(jnp.int32, (BQ, L), 1)
    key = jax.lax.bitcast_convert_type(d2, jnp.int32)
    key = (key & jnp.int32(-2048)) | iota
    sel = jnp.zeros((BQ, L), jnp.float32)
    for t in range(16):
        v = jnp.min(key, axis=1, keepdims=True)
        pick = key == v
        sel = jnp.where(pick, jnp.float32(1.0), sel)
        key = jnp.where(pick, jnp.int32(0x7FFFFFFF), key)
        if t == 7:
            m8_o[...] = sel.astype(jnp.bfloat16)
    m16_o[...] = sel.astype(jnp.bfloat16)


def _attn_kernel(qp, kT, vT, m8, m16, o8_o, o16_o):
    scale = jnp.float32(1.0 / np.sqrt(DH))
    sidx = pl.program_id(1)
    cidx = 1 - sidx
    q = qp[0]
    m8f = m8[...]
    m16f = m16[...]
    bf = jnp.bfloat16
    ct0 = (((1,), (0,)), ((), ()))
    ct1 = (((1,), (1,)), ((), ()))
    outs8 = []
    outs16 = []
    for h in range(H):
        sl = slice(h * DH, (h + 1) * DH)
        qh = (q[:, sl] * scale).astype(bf)
        s_self = jax.lax.dot_general(qh, kT[sidx, sl, :], ct0,
                                     preferred_element_type=jnp.float32)
        s_cross = jax.lax.dot_general(qh, kT[cidx, sl, :], ct0,
                                      preferred_element_type=jnp.float32)
        e_self = jnp.exp(s_self).astype(bf)
        e_cross = jnp.exp(s_cross).astype(bf)
        uz_c = jax.lax.dot_general(e_cross, vT[cidx, h], ct1,
                                   preferred_element_type=jnp.float32)
        e8 = e_self * m8f
        e16 = e_self * m16f
        vs_h = vT[sidx, h]
        uz8 = uz_c + jax.lax.dot_general(e8, vs_h, ct1,
                                         preferred_element_type=jnp.float32)
        uz16 = uz_c + jax.lax.dot_general(e16, vs_h, ct1,
                                          preferred_element_type=jnp.float32)
        outs8.append(uz8[:, 0:DH] / uz8[:, DH:DH + 1])
        outs16.append(uz16[:, 0:DH] / uz16[:, DH:DH + 1])
    o8_o[0] = jnp.concatenate(outs8, axis=1)
    o16_o[0] = jnp.concatenate(outs16, axis=1)


def _mix_kernel(g8, g16, s8, s16, geo_p, sem_p, rsa_out,
                Wo, bo, ln1g, ln1b, ln2g, ln2b,
                Wf1, bf1, f1g, f1b, Wf2, bf2, f2g, f2b,
                mix, out_o):
    mv = mix[...]
    swg = mv[0:1, 0:2]
    sws = mv[0:1, 2:4]
    wg = jnp.exp(swg - jnp.max(swg))
    wg = wg / jnp.sum(wg)
    ws = jnp.exp(sws - jnp.max(sws))
    ws = ws / jnp.sum(ws)
    wg0, wg1 = wg[0:1, 0:1], wg[0:1, 1:2]
    ws0, ws1 = ws[0:1, 0:1], ws[0:1, 1:2]
    alpha_g = mv[0:1, 4:5]
    beta_g = mv[0:1, 5:6]
    alpha_s = mv[0:1, 6:7]
    beta_s = mv[0:1, 7:8]

    Wo_m = Wo[...]
    bo_m = bo[...]
    geo_attn = (wg0 * g8[0] + wg1 * g16[0]) @ Wo_m + bo_m
    sem_attn = (ws0 * s8[0] + ws1 * s16[0]) @ Wo_m + bo_m
    geo_out = _ln(alpha_g * geo_p[...] + beta_g * geo_attn, ln1g[...], ln1b[...])
    sem_out = _ln(alpha_s * sem_p[...] + beta_s * sem_attn, ln2g[...], ln2b[...])
    W1 = Wf1[...]
    h1 = (geo_out @ W1[0:D, :] + sem_out @ W1[D:2 * D, :]
          + rsa_out[...] @ W1[2 * D:3 * D, :] + bf1[...])
    x = _lrelu(_ln(h1, f1g[...], f1b[...]))
    x = _lrelu(_ln(x @ Wf2[...] + bf2[...], f2g[...], f2b[...]))
    out_o[...] = x


def _row(v):
    return v.reshape(1, -1)


def _full_spec(shape):
    n = len(shape)
    return pl.BlockSpec(shape, lambda *_, _n=n: (0,) * _n)


def _blk_spec(cols):
    return pl.BlockSpec((BQ, cols), lambda i: (i, 0))


@jax.jit
def kernel(geo_feat, sem_feat, rsa_feat, pos, params):
    p = params
    f32 = jnp.float32

    posT = jnp.zeros((8, L), f32).at[0:3, :].set(pos.T)
    pos_pad = jnp.zeros((L, 8), f32).at[:, 0:3].set(pos)

    S = np.zeros((D, 128), np.float32)
    B = np.zeros((128, D), np.float32)
    for h in range(H):
        S[h * DH:(h + 1) * DH, h] = 1.0 / DH
        B[h, h * DH:(h + 1) * DH] = 1.0
    S = jnp.asarray(S)
    B = jnp.asarray(B)

    mix = jnp.zeros((1, 128), f32)
    mix = mix.at[0, 0:2].set(p['sw_g'])
    mix = mix.at[0, 2:4].set(p['sw_s'])
    mix = mix.at[0, 4].set(p['alpha_g'])
    mix = mix.at[0, 5].set(p['beta_g'])
    mix = mix.at[0, 6].set(p['alpha_s'])
    mix = mix.at[0, 7].set(p['beta_s'])

    LD = jax.ShapeDtypeStruct((L, D), f32)

    proj_in = [geo_feat, sem_feat, rsa_feat,
               p['Wg'], _row(p['bg']), _row(p['g_g']), _row(p['g_b']),
               p['Ws'], _row(p['bs']), _row(p['s_g']), _row(p['s_b']),
               p['Wr'], _row(p['br']), _row(p['r_g']), _row(p['r_b']),
               p['Wqg'], _row(p['bqg']), p['Wqs'], _row(p['bqs']),
               p['Wk'], _row(p['bk']), p['Wv'], _row(p['bv']),
               p['Wt'], _row(p['bt']), _row(p['t_g']), _row(p['t_b']),
               S, B]
    proj_specs = ([_blk_spec(GEO_DIM), _blk_spec(SEM_DIM), _blk_spec(RSA_DIM)]
                  + [_full_spec(a.shape) for a in proj_in[3:]])
    bfl = jnp.bfloat16
    geo_p, sem_p, Qp, KT, VT, rsa_out = pl.pallas_call(
        _proj_kernel,
        grid=(NBLK,),
        in_specs=proj_specs,
        out_specs=[_blk_spec(D), _blk_spec(D),
                   pl.BlockSpec((2, BQ, D), lambda i: (0, i, 0)),
                   pl.BlockSpec((2, D, BQ), lambda i: (0, 0, i)),
                   pl.BlockSpec((2, H, DH + 8, BQ), lambda i: (0, 0, 0, i)),
                   _blk_spec(D)],
        out_shape=[LD, LD,
                   jax.ShapeDtypeStruct((2, L, D), f32),
                   jax.ShapeDtypeStruct((2, D, L), bfl),
                   jax.ShapeDtypeStruct((2, H, DH + 8, L), bfl),
                   LD],
    )(*proj_in)

    m8, m16 = pl.pallas_call(
        _knn_kernel,
        grid=(NBLK,),
        in_specs=[_blk_spec(8), _full_spec((8, L))],
        out_specs=[_blk_spec(L)] * 2,
        out_shape=[jax.ShapeDtypeStruct((L, L), jnp.bfloat16)] * 2,
    )(pos_pad, posT)

    qspec = pl.BlockSpec((1, BQ, D), lambda qi, s: (s, qi, 0))
    mspec = pl.BlockSpec((BQ, L), lambda qi, s: (qi, 0))
    o8p, o16p = pl.pallas_call(
        _attn_kernel,
        grid=(NBLK, 2),
        in_specs=[qspec, _full_spec((2, D, L)), _full_spec((2, H, DH + 8, L)),
                  mspec, mspec],
        out_specs=[qspec, qspec],
        out_shape=[jax.ShapeDtypeStruct((2, L, D), f32)] * 2,
    )(Qp, KT, VT, m8, m16)

    gsp = pl.BlockSpec((1, BQ, D), lambda i: (0, i, 0))
    ssp = pl.BlockSpec((1, BQ, D), lambda i: (1, i, 0))
    mix_in = [o8p, o16p, o8p, o16p, geo_p, sem_p, rsa_out,
              p['Wo'], _row(p['bo']),
              _row(p['ln1_g']), _row(p['ln1_b']),
              _row(p['ln2_g']), _row(p['ln2_b']),
              p['Wf1'], _row(p['bf1']), _row(p['f1_g']), _row(p['f1_b']),
              p['Wf2'], _row(p['bf2']), _row(p['f2_g']), _row(p['f2_b']),
              mix]
    mix_specs = ([gsp, gsp, ssp, ssp] + [_blk_spec(D)] * 3
                 + [_full_spec(a.shape) for a in mix_in[7:]])
    out = pl.pallas_call(
        _mix_kernel,
        grid=(NBLK,),
        in_specs=mix_specs,
        out_specs=_blk_spec(D),
        out_shape=LD,
    )(*mix_in)
    return out
